# Initial kernel scaffold; baseline (speedup 1.0000x reference)
import jax
import jax.numpy as jnp
from jax.experimental import pallas as pl


def kernel(x_nchw, w_oihw, gamma, beta):
    raise NotImplementedError("write your pallas kernel here")


# speedup vs baseline: 7.8727x; 7.8727x over previous
import functools

import jax
import jax.numpy as jnp
from jax.experimental import pallas as pl
from jax.experimental.pallas import tpu as pltpu


def _conv_stats_kernel(x_ref, w_ref, y_ref, stats_ref, *, W, KH, KW, pad):
    x = x_ref[0]
    cin, hw = x.shape
    ext = W * (KH - 1 - pad) + (KW - 1 - pad)
    lo = W * pad + pad
    xe = jnp.pad(x, ((0, 0), (lo, ext)))
    wcol = jax.lax.broadcasted_iota(jnp.int32, (1, hw), 1) % W
    taps = []
    for kh in range(KH):
        for kw in range(KW):
            d = (kh - pad) * W + (kw - pad)
            t = xe[:, lo + d:lo + d + hw]
            if kw - pad < 0:
                t = jnp.where(wcol >= pad - kw, t, 0.0)
            elif kw - pad > 0:
                t = jnp.where(wcol < W - (kw - pad), t, 0.0)
            taps.append(t)
    patches = jnp.concatenate(taps, axis=0)
    y = jnp.dot(w_ref[...], patches, preferred_element_type=jnp.float32)
    y_ref[0] = y.astype(y_ref.dtype)
    s = jnp.sum(y, axis=1, keepdims=True)
    ss = jnp.sum(y * y, axis=1, keepdims=True)
    stats_ref[0] = jnp.concatenate([s, ss], axis=1)


def _bn_relu_kernel(y_ref, sc_ref, sh_ref, o_ref):
    y = y_ref[0].astype(jnp.float32)
    o_ref[0] = jnp.maximum(y * sc_ref[...] + sh_ref[...], 0.0).astype(o_ref.dtype)


@functools.partial(jax.jit, static_argnames=())
def kernel(x_nchw, w_oihw, gamma, beta):
    eps = 1e-5
    N, Cin, H, W = x_nchw.shape
    Cout, _, KH, KW = w_oihw.shape
    pad = 1
    HW = H * W
    K = KH * KW * Cin

    x = x_nchw.reshape(N, Cin, HW)
    wflat = jnp.transpose(w_oihw, (0, 2, 3, 1)).reshape(Cout, K)

    conv_body = functools.partial(_conv_stats_kernel, W=W, KH=KH, KW=KW, pad=pad)
    flops1 = 2 * N * HW * K * Cout
    bytes1 = x.size * 4 + wflat.size * 4 + N * Cout * HW * 2 + N * Cout * 2 * 4
    y, stats = pl.pallas_call(
        conv_body,
        out_shape=(
            jax.ShapeDtypeStruct((N, Cout, HW), jnp.bfloat16),
            jax.ShapeDtypeStruct((N, Cout, 2), jnp.float32),
        ),
        grid=(N,),
        in_specs=[
            pl.BlockSpec((1, Cin, HW), lambda n: (n, 0, 0)),
            pl.BlockSpec((Cout, K), lambda n: (0, 0)),
        ],
        out_specs=(
            pl.BlockSpec((1, Cout, HW), lambda n: (n, 0, 0)),
            pl.BlockSpec((1, Cout, 2), lambda n: (n, 0, 0)),
        ),
        compiler_params=pltpu.CompilerParams(
            dimension_semantics=("parallel",),
            vmem_limit_bytes=48 * 1024 * 1024,
        ),
        cost_estimate=pl.CostEstimate(
            flops=flops1, transcendentals=0, bytes_accessed=bytes1
        ),
    )(x, wflat)

    totals = jnp.sum(stats, axis=0)
    count = N * HW
    mean = totals[:, 0] / count
    var = jnp.maximum(totals[:, 1] / count - mean * mean, 0.0)
    scale = gamma.astype(jnp.float32) * jax.lax.rsqrt(var + eps)
    shift = beta.astype(jnp.float32) - mean * scale

    bytes2 = N * Cout * HW * (2 + 4) + 2 * Cout * 4
    out = pl.pallas_call(
        _bn_relu_kernel,
        out_shape=jax.ShapeDtypeStruct((N, Cout, HW), x_nchw.dtype),
        grid=(N,),
        in_specs=[
            pl.BlockSpec((1, Cout, HW), lambda n: (n, 0, 0)),
            pl.BlockSpec((Cout, 1), lambda n: (0, 0)),
            pl.BlockSpec((Cout, 1), lambda n: (0, 0)),
        ],
        out_specs=pl.BlockSpec((1, Cout, HW), lambda n: (n, 0, 0)),
        compiler_params=pltpu.CompilerParams(
            dimension_semantics=("parallel",),
            vmem_limit_bytes=32 * 1024 * 1024,
        ),
        cost_estimate=pl.CostEstimate(
            flops=2 * N * Cout * HW, transcendentals=0, bytes_accessed=bytes2
        ),
    )(y, scale.reshape(Cout, 1), shift.reshape(Cout, 1))

    return out.reshape(N, Cout, H, W)
